# baseline (speedup 1.0000x reference)
import functools

import numpy as np
import jax
import jax.numpy as jnp
from jax import lax
from jax.experimental import pallas as pl
from jax.experimental.pallas import tpu as pltpu
from jax.experimental.pallas import tpu_sc as plsc

B = 4096
ND = 13
NF = 26
NFE = NF + 1
MH = 10000
H = 160
ED = 64

NP = NF // 2
NQ = 6
NG = NQ + 1
QW = 4 * ED
QWI = QW // 2

NC, NS = 2, 16
NW = NC * NS
CH = 128


@functools.partial(
    pl.kernel,
    mesh=plsc.VectorSubcoreMesh(core_axis_name="c", subcore_axis_name="s"),
    out_type=jax.ShapeDtypeStruct((NG * B, QWI), jnp.int32),
    scratch_types=[
        pltpu.VMEM((CH,), jnp.int32),
        pltpu.VMEM((CH,), jnp.int32),
        pltpu.VMEM((CH,), jnp.int32),
        pltpu.VMEM((CH, QWI), jnp.int32),
        pltpu.VMEM((CH, QWI), jnp.int32),
        pltpu.SemaphoreType.DMA,
        pltpu.SemaphoreType.DMA,
    ],
)
def _sc_gather(tabq_ref, tabp_ref, ids_ref, out_ref, ids_v, idxA, idxB,
               buf0, buf1, sem0, sem1):
    wid = lax.axis_index("s") * NC + lax.axis_index("c")
    pltpu.sync_copy(ids_ref.at[pl.ds(wid * CH, CH)], ids_v)
    idxs = (idxA, idxB)
    bufs = (buf0, buf1)
    sems = (sem0, sem1)
    cps = []
    for c in range(NG):
        idxs[c % 2][...] = ids_v[...] + (c * MH if c < NQ else 0)
        tab = tabq_ref if c < NQ else tabp_ref
        cps.append(pltpu.async_copy(tab.at[idxs[c % 2]],
                                    bufs[c % 2], sems[c % 2]))
        if c >= 1:
            cps[c - 1].wait()
            pltpu.sync_copy(
                bufs[(c - 1) % 2],
                out_ref.at[pl.ds(((c - 1) * NW + wid) * CH, CH)])
    cps[NG - 1].wait()
    pltpu.sync_copy(bufs[(NG - 1) % 2],
                    out_ref.at[pl.ds(((NG - 1) * NW + wid) * CH, CH)])


TBLK = 2048


def _quad_body(t_ref, w_ref, b_ref, o_ref):
    ones_row = jnp.ones((1, H), jnp.float32)
    halves = []
    for p in range(2):
        for k in range(2):
            r = t_ref[p, k]
            n2 = jnp.dot(ones_row, r * r,
                         preferred_element_type=jnp.float32)
            s = jnp.where(n2 > 1.0, lax.rsqrt(n2), 1.0)
            halves.append(
                jnp.dot(w_ref[p, k], r,
                        preferred_element_type=jnp.float32) * s
                + b_ref[p, k]
            )
    out = jnp.transpose(jnp.concatenate(halves, axis=0))
    o_ref[0] = out.astype(jnp.bfloat16)


def _pair_body(t_ref, w_ref, b_ref, o_ref):
    ones_row = jnp.ones((1, H), jnp.float32)
    halves = []
    for k in range(2):
        r = t_ref[0, k]
        n2 = jnp.dot(ones_row, r * r, preferred_element_type=jnp.float32)
        s = jnp.where(n2 > 1.0, lax.rsqrt(n2), 1.0)
        halves.append(
            jnp.dot(w_ref[0, k], r, preferred_element_type=jnp.float32) * s
            + b_ref[0, k]
        )
    halves.append(jnp.zeros((2 * ED, TBLK), jnp.float32))
    out = jnp.transpose(jnp.concatenate(halves, axis=0))
    o_ref[0] = out.astype(jnp.bfloat16)


def _quad_transform(tablesT4, Wp4, bp4):
    return pl.pallas_call(
        _quad_body,
        grid=(NQ, pl.cdiv(MH, TBLK)),
        in_specs=[
            pl.BlockSpec((2, 2, H, TBLK), lambda q, i: (q, 0, 0, i)),
            pl.BlockSpec((2, 2, ED, H), lambda q, i: (q, 0, 0, 0)),
            pl.BlockSpec((2, 2, ED, 1), lambda q, i: (q, 0, 0, 0)),
        ],
        out_specs=pl.BlockSpec((1, TBLK, QW), lambda q, i: (q, i, 0)),
        out_shape=jax.ShapeDtypeStruct((NQ, MH, QW), jnp.bfloat16),
    )(tablesT4, Wp4, bp4)


def _pair_transform(tablesT4, Wp4, bp4):
    return pl.pallas_call(
        _pair_body,
        grid=(1, pl.cdiv(MH, TBLK)),
        in_specs=[
            pl.BlockSpec((1, 2, H, TBLK), lambda q, i: (NP - 1, 0, 0, i)),
            pl.BlockSpec((1, 2, ED, H), lambda q, i: (NP - 1, 0, 0, 0)),
            pl.BlockSpec((1, 2, ED, 1), lambda q, i: (NP - 1, 0, 0, 0)),
        ],
        out_specs=pl.BlockSpec((1, TBLK, QW), lambda q, i: (q, i, 0)),
        out_shape=jax.ShapeDtypeStruct((1, MH, QW), jnp.bfloat16),
    )(tablesT4, Wp4, bp4)


BLK = 256
NPAIR = NFE * (NFE - 1) // 2


def _bat_body(g_ref, dt_ref, wd_ref, bd_ref, wo_ref, bo_ref, o_ref,
              ecat_ref, gt_ref):
    e0t = (
        jnp.dot(wd_ref[...], dt_ref[...], preferred_element_type=jnp.float32)
        + bd_ref[...]
    )
    ets = [e0t]
    for q in range(NG):
        tp = jnp.transpose(g_ref[q]).astype(jnp.float32)
        nfeat = 4 if q < NQ else 2
        for f in range(nfeat):
            ets.append(tp[f * ED:(f + 1) * ED])
    for f in range(NFE):
        ecat_ref[f * ED:(f + 1) * ED, :] = ets[f]
    k = 0
    for i in range(NFE):
        for j in range(i + 1, NFE):
            gt_ref[k, :] = jnp.sum(ets[i] * ets[j], axis=0)
            k += 1
    w1 = wo_ref[:, : NFE * ED]
    w2 = wo_ref[:, NFE * ED:]
    o_ref[...] = (
        jnp.dot(w1, ecat_ref[...], preferred_element_type=jnp.float32)
        + jnp.dot(w2, gt_ref[...], preferred_element_type=jnp.float32)
        + bo_ref[...]
    )


def _batch_compute(g3, denseT, Wd, bd2, Wo, bo2):
    return pl.pallas_call(
        _bat_body,
        grid=(B // BLK,),
        in_specs=[
            pl.BlockSpec((NG, BLK, QW), lambda i: (0, i, 0)),
            pl.BlockSpec((ND, BLK), lambda i: (0, i)),
            pl.BlockSpec((ED, ND), lambda i: (0, 0)),
            pl.BlockSpec((ED, 1), lambda i: (0, 0)),
            pl.BlockSpec((ED, NFE * ED + NPAIR), lambda i: (0, 0)),
            pl.BlockSpec((ED, 1), lambda i: (0, 0)),
        ],
        out_specs=pl.BlockSpec((ED, BLK), lambda i: (0, i)),
        out_shape=jax.ShapeDtypeStruct((ED, B), jnp.float32),
        scratch_shapes=[
            pltpu.VMEM((NFE * ED, BLK), jnp.float32),
            pltpu.VMEM((NPAIR, BLK), jnp.float32),
        ],
    )(g3, denseT, Wd, bd2, Wo, bo2)


def kernel(dense, id_list, offsets, tables, Wd, bd, Wp, bp, Wo, bo):
    ids = (id_list.astype(jnp.int32)) % MH
    tablesT4 = jnp.transpose(tables, (0, 2, 1)).reshape(NP, 2, H, MH)
    Wp4 = Wp.reshape(NP, 2, ED, H)
    bp4 = bp.reshape(NP, 2, ED, 1)
    tq = _quad_transform(tablesT4, Wp4, bp4)
    tp_ = _pair_transform(tablesT4, Wp4, bp4)
    tqi = lax.bitcast_convert_type(
        tq.reshape(NQ * MH, QWI, 2), jnp.int32)
    tpi = lax.bitcast_convert_type(
        tp_.reshape(MH, QWI, 2), jnp.int32)
    gi = _sc_gather(tqi, tpi, ids)
    g = lax.bitcast_convert_type(gi, jnp.bfloat16)
    g3 = g.reshape(NG, B, QW)
    outT = _batch_compute(g3, dense.T, Wd, bd[:, None], Wo, bo[:, None])
    return outT.T


# speedup vs baseline: 5.9967x; 5.9967x over previous
import functools

import numpy as np
import jax
import jax.numpy as jnp
from jax import lax
from jax.experimental import pallas as pl
from jax.experimental.pallas import tpu as pltpu
from jax.experimental.pallas import tpu_sc as plsc

B = 4096
ND = 13
NF = 26
NFE = NF + 1
MH = 10000
H = 160
ED = 64

NP = NF // 2
NQ = 6
NG = NQ + 1
QW = 4 * ED
QWI = QW // 2

NC, NS = 2, 16
NW = NC * NS
CH = 128


@functools.partial(
    pl.kernel,
    mesh=plsc.VectorSubcoreMesh(core_axis_name="c", subcore_axis_name="s"),
    out_type=jax.ShapeDtypeStruct((NG * B, QWI), jnp.int32),
    scratch_types=[
        pltpu.VMEM((CH,), jnp.int32),
        pltpu.VMEM((CH,), jnp.int32),
        pltpu.VMEM((CH,), jnp.int32),
        pltpu.VMEM((CH, QWI), jnp.int32),
        pltpu.VMEM((CH, QWI), jnp.int32),
        pltpu.SemaphoreType.DMA,
        pltpu.SemaphoreType.DMA,
    ],
)
def _sc_gather(tabq_ref, tabp_ref, ids_ref, out_ref, ids_v, idxA, idxB,
               buf0, buf1, sem0, sem1):
    wid = lax.axis_index("s") * NC + lax.axis_index("c")
    pltpu.sync_copy(ids_ref.at[pl.ds(wid * CH, CH)], ids_v)
    idxs = (idxA, idxB)
    bufs = (buf0, buf1)
    sems = (sem0, sem1)
    cps = []
    for c in range(NG):
        idxs[c % 2][...] = ids_v[...] + (c * MH if c < NQ else 0)
        tab = tabq_ref if c < NQ else tabp_ref
        cps.append(pltpu.async_copy(tab.at[idxs[c % 2]],
                                    bufs[c % 2], sems[c % 2]))
        if c >= 1:
            cps[c - 1].wait()
            pltpu.sync_copy(
                bufs[(c - 1) % 2],
                out_ref.at[pl.ds(((c - 1) * NW + wid) * CH, CH)])
    cps[NG - 1].wait()
    pltpu.sync_copy(bufs[(NG - 1) % 2],
                    out_ref.at[pl.ds(((NG - 1) * NW + wid) * CH, CH)])


TBLK = 2048


def _bf16_bits(x):
    y = x.astype(jnp.bfloat16).astype(jnp.float32)
    return lax.bitcast_convert_type(y, jnp.int32)


def _quad_body(t_ref, w_ref, b_ref, o_ref):
    ones_row = jnp.ones((1, H), jnp.float32)
    halves = []
    for p in range(2):
        for k in range(2):
            r = t_ref[p, k]
            n2 = jnp.dot(ones_row, r * r,
                         preferred_element_type=jnp.float32)
            s = jnp.where(n2 > 1.0, lax.rsqrt(n2), 1.0)
            halves.append(
                jnp.dot(w_ref[p, k], r,
                        preferred_element_type=jnp.float32) * s
                + b_ref[p, k]
            )
    h01 = jnp.concatenate(halves[:2], axis=0)
    h23 = jnp.concatenate(halves[2:], axis=0)
    word = _bf16_bits(h23) | lax.shift_right_logical(_bf16_bits(h01), 16)
    o_ref[0] = jnp.transpose(word)


def _pair_body(t_ref, w_ref, b_ref, o_ref):
    ones_row = jnp.ones((1, H), jnp.float32)
    halves = []
    for k in range(2):
        r = t_ref[0, k]
        n2 = jnp.dot(ones_row, r * r, preferred_element_type=jnp.float32)
        s = jnp.where(n2 > 1.0, lax.rsqrt(n2), 1.0)
        halves.append(
            jnp.dot(w_ref[0, k], r, preferred_element_type=jnp.float32) * s
            + b_ref[0, k]
        )
    h01 = jnp.concatenate(halves, axis=0)
    word = lax.shift_right_logical(_bf16_bits(h01), 16)
    o_ref[0] = jnp.transpose(word)


def _quad_transform(tablesT4, Wp4, bp4):
    return pl.pallas_call(
        _quad_body,
        grid=(NQ, pl.cdiv(MH, TBLK)),
        in_specs=[
            pl.BlockSpec((2, 2, H, TBLK), lambda q, i: (q, 0, 0, i)),
            pl.BlockSpec((2, 2, ED, H), lambda q, i: (q, 0, 0, 0)),
            pl.BlockSpec((2, 2, ED, 1), lambda q, i: (q, 0, 0, 0)),
        ],
        out_specs=pl.BlockSpec((1, TBLK, QWI), lambda q, i: (q, i, 0)),
        out_shape=jax.ShapeDtypeStruct((NQ, MH, QWI), jnp.int32),
    )(tablesT4, Wp4, bp4)


def _pair_transform(tablesT4, Wp4, bp4):
    return pl.pallas_call(
        _pair_body,
        grid=(1, pl.cdiv(MH, TBLK)),
        in_specs=[
            pl.BlockSpec((1, 2, H, TBLK), lambda q, i: (NP - 1, 0, 0, i)),
            pl.BlockSpec((1, 2, ED, H), lambda q, i: (NP - 1, 0, 0, 0)),
            pl.BlockSpec((1, 2, ED, 1), lambda q, i: (NP - 1, 0, 0, 0)),
        ],
        out_specs=pl.BlockSpec((1, TBLK, QWI), lambda q, i: (q, i, 0)),
        out_shape=jax.ShapeDtypeStruct((1, MH, QWI), jnp.int32),
    )(tablesT4, Wp4, bp4)


BLK = 256
NPAIR = NFE * (NFE - 1) // 2


def _bat_body(g_ref, dt_ref, wd_ref, bd_ref, wo_ref, bo_ref, o_ref,
              ecat_ref, gt_ref):
    e0t = (
        jnp.dot(wd_ref[...], dt_ref[...], preferred_element_type=jnp.float32)
        + bd_ref[...]
    )
    ets = [e0t]
    for q in range(NG):
        w = jnp.transpose(g_ref[q])
        lo = lax.bitcast_convert_type(lax.shift_left(w, 16), jnp.float32)
        ets.append(lo[:ED])
        ets.append(lo[ED:])
        if q < NQ:
            hi = lax.bitcast_convert_type(
                jnp.bitwise_and(w, jnp.int32(-65536)), jnp.float32)
            ets.append(hi[:ED])
            ets.append(hi[ED:])
    for f in range(NFE):
        ecat_ref[f * ED:(f + 1) * ED, :] = ets[f]
    k = 0
    for i in range(NFE):
        for j in range(i + 1, NFE):
            gt_ref[k, :] = jnp.sum(ets[i] * ets[j], axis=0)
            k += 1
    w1 = wo_ref[:, : NFE * ED]
    w2 = wo_ref[:, NFE * ED:]
    o_ref[...] = (
        jnp.dot(w1, ecat_ref[...], preferred_element_type=jnp.float32)
        + jnp.dot(w2, gt_ref[...], preferred_element_type=jnp.float32)
        + bo_ref[...]
    )


def _batch_compute(g3, denseT, Wd, bd2, Wo, bo2):
    return pl.pallas_call(
        _bat_body,
        grid=(B // BLK,),
        in_specs=[
            pl.BlockSpec((NG, BLK, QWI), lambda i: (0, i, 0)),
            pl.BlockSpec((ND, BLK), lambda i: (0, i)),
            pl.BlockSpec((ED, ND), lambda i: (0, 0)),
            pl.BlockSpec((ED, 1), lambda i: (0, 0)),
            pl.BlockSpec((ED, NFE * ED + NPAIR), lambda i: (0, 0)),
            pl.BlockSpec((ED, 1), lambda i: (0, 0)),
        ],
        out_specs=pl.BlockSpec((ED, BLK), lambda i: (0, i)),
        out_shape=jax.ShapeDtypeStruct((ED, B), jnp.float32),
        scratch_shapes=[
            pltpu.VMEM((NFE * ED, BLK), jnp.float32),
            pltpu.VMEM((NPAIR, BLK), jnp.float32),
        ],
    )(g3, denseT, Wd, bd2, Wo, bo2)


def kernel(dense, id_list, offsets, tables, Wd, bd, Wp, bp, Wo, bo):
    ids = (id_list.astype(jnp.int32)) % MH
    tablesT4 = jnp.transpose(tables, (0, 2, 1)).reshape(NP, 2, H, MH)
    Wp4 = Wp.reshape(NP, 2, ED, H)
    bp4 = bp.reshape(NP, 2, ED, 1)
    tqi = _quad_transform(tablesT4, Wp4, bp4)
    tpi = _pair_transform(tablesT4, Wp4, bp4)
    gi = _sc_gather(tqi.reshape(NQ * MH, QWI),
                    tpi.reshape(MH, QWI), ids)
    g3 = gi.reshape(NG, B, QWI)
    outT = _batch_compute(g3, dense.T, Wd, bd[:, None], Wo, bo[:, None])
    return outT.T
